# baseline (speedup 1.0000x reference)
import functools

import jax
import jax.numpy as jnp
from jax import lax
from jax.experimental import pallas as pl
from jax.experimental.pallas import tpu as pltpu
from jax.experimental.pallas import tpu_sc as plsc

BATCH = 16384
NUM_WORKERS = 32
BPW = BATCH // NUM_WORKERS
CHUNK = 128
NCHUNK = BPW // CHUNK
D_MEM, D_PARTY, D_STATE = 32, 16, 16
D_OUT = D_MEM + D_PARTY + D_STATE


def _sc_body(member_hbm, state_hbm, party_hbm,
             mtab_hbm, stab_hbm, ptab_hbm, out_hbm,
             midx_v, sidx_v, pidx_v, mrows_v, prows_v, srows_v,
             msem, psem, ssem):
    wid = lax.axis_index("s") * 2 + lax.axis_index("c")
    base = wid * BPW
    row0 = wid * NCHUNK

    pltpu.sync_copy(member_hbm.at[pl.ds(row0, NCHUNK)], midx_v)
    pltpu.sync_copy(state_hbm.at[pl.ds(row0, NCHUNK)], sidx_v)
    pltpu.sync_copy(party_hbm.at[pl.ds(row0, NCHUNK)], pidx_v)

    copies = []
    for j in range(NCHUNK):
        rows = pl.ds(j * CHUNK, CHUNK)
        copies.append(pltpu.async_copy(
            mtab_hbm.at[midx_v.at[j]], mrows_v.at[rows], msem))
        copies.append(pltpu.async_copy(
            ptab_hbm.at[pidx_v.at[j]], prows_v.at[rows], psem))
        copies.append(pltpu.async_copy(
            stab_hbm.at[sidx_v.at[j]], srows_v.at[rows], ssem))
    for c in copies:
        c.wait()

    rows = pl.ds(base, BPW)
    pltpu.sync_copy(mrows_v, out_hbm.at[rows, pl.ds(0, D_MEM)])
    pltpu.sync_copy(prows_v, out_hbm.at[rows, pl.ds(D_MEM, D_PARTY)])
    pltpu.sync_copy(srows_v, out_hbm.at[rows, pl.ds(D_MEM + D_PARTY, D_STATE)])


@jax.jit
def _mem_encoder_sc(member, state, party, member_table, state_table, party_table):
    mesh = plsc.VectorSubcoreMesh(core_axis_name="c", subcore_axis_name="s")
    k = functools.partial(
        pl.kernel,
        out_type=jax.ShapeDtypeStruct((BATCH, D_OUT), jnp.float32),
        mesh=mesh,
        scratch_types=[
            pltpu.VMEM((NCHUNK, CHUNK), jnp.int32),
            pltpu.VMEM((NCHUNK, CHUNK), jnp.int32),
            pltpu.VMEM((NCHUNK, CHUNK), jnp.int32),
            pltpu.VMEM((BPW, D_MEM), jnp.float32),
            pltpu.VMEM((BPW, D_PARTY), jnp.float32),
            pltpu.VMEM((BPW, D_STATE), jnp.float32),
            pltpu.SemaphoreType.DMA,
            pltpu.SemaphoreType.DMA,
            pltpu.SemaphoreType.DMA,
        ],
        compiler_params=pltpu.CompilerParams(use_tc_tiling_on_sc=False),
    )(_sc_body)
    member2d = member.astype(jnp.int32).reshape(BATCH // CHUNK, CHUNK)
    state2d = state.astype(jnp.int32).reshape(BATCH // CHUNK, CHUNK)
    party2d = party.astype(jnp.int32).reshape(BATCH // CHUNK, CHUNK)
    return k(member2d, state2d, party2d, member_table, state_table, party_table)


def kernel(member, state, party, member_table, state_table, party_table):
    return _mem_encoder_sc(member, state, party,
                           member_table, state_table, party_table)
